# baseline (speedup 1.0000x reference)
import jax
import jax.numpy as jnp
from jax import lax
from jax.experimental import pallas as pl
from jax.experimental.pallas import tpu as pltpu
from jax.experimental.pallas import tpu_sc as plsc

N_NODES = 2048
N_RELS = 8
N_CLASSES = 2048
EMB = 1600
W = 16
RW = N_RELS * W
T = 32768
SEG = N_RELS * N_NODES

NT = 16
TP = T // NT
VC = TP // 16
CH = TP // 128
HROWS = N_NODES // NT
OROWS = SEG // NT


def _mm1_body(x_ref, w_ref, o_ref):
    o_ref[...] = jnp.dot(x_ref[...], w_ref[...],
                         preferred_element_type=jnp.float32)


def _mm1(x, w1cat):
    return pl.pallas_call(
        _mm1_body,
        grid=(8,),
        in_specs=[
            pl.BlockSpec((N_NODES // 8, EMB), lambda i: (i, 0)),
            pl.BlockSpec((EMB, RW), lambda i: (0, 0)),
        ],
        out_specs=pl.BlockSpec((N_NODES // 8, RW), lambda i: (i, 0)),
        out_shape=jax.ShapeDtypeStruct((N_NODES, RW), jnp.float32),
    )(x, w1cat)


def _mm2_body(a_ref, b_ref, bias_ref, o_ref):
    o_ref[...] = (jnp.dot(a_ref[...], b_ref[...],
                          preferred_element_type=jnp.float32)
                  + bias_ref[...])


def _mm2(a, b, bias2_row):
    return pl.pallas_call(
        _mm2_body,
        grid=(8,),
        in_specs=[
            pl.BlockSpec((N_NODES // 8, RW), lambda i: (i, 0)),
            pl.BlockSpec((RW, N_CLASSES), lambda i: (0, 0)),
            pl.BlockSpec((1, N_CLASSES), lambda i: (0, 0)),
        ],
        out_specs=pl.BlockSpec((N_NODES // 8, N_CLASSES), lambda i: (i, 0)),
        out_shape=jax.ShapeDtypeStruct((N_NODES, N_CLASSES), jnp.float32),
    )(a, b, bias2_row)


def _sc_cnt_body(s3, p3, ones_h, zero_h, cnt_out,
                 sv, pv, rvidx, ones, zrow, cbuf, cnt_sh, semz, semc):
    wid = lax.axis_index("sx")
    pltpu.sync_copy(s3.at[wid], sv)
    pltpu.sync_copy(p3.at[wid], pv)
    pltpu.sync_copy(ones_h, ones)
    pltpu.sync_copy(zero_h, zrow)

    zps = []
    for j in range(OROWS // 128):
        zps.append(pltpu.async_copy(
            zrow, cnt_sh.at[pl.ds(wid * OROWS + j * 128, 128)], semz))

    @plsc.parallel_loop(0, VC, unroll=4)
    def _didx(c):
        rvidx[c >> 3, pl.ds((c & 7) * 16, 16)] = pv[c] * N_NODES + sv[c]

    for cp in zps:
        cp.wait()
    plsc.subcore_barrier()

    cnt_ps = [pltpu.async_copy(ones, cnt_sh.at[rvidx.at[j]], semc, add=True)
              for j in range(CH)]
    for cp in cnt_ps:
        cp.wait()
    plsc.subcore_barrier()

    ob = wid * OROWS
    pltpu.sync_copy(cnt_sh.at[pl.ds(ob, OROWS)], cbuf)

    @plsc.parallel_loop(0, OROWS, unroll=8)
    def _inv(i):
        cbuf[i] = 1.0 / cbuf[i]

    pltpu.sync_copy(cbuf, cnt_out.at[pl.ds(ob, OROWS)])


def _sc_cnt(s3, p3, ones_h, zero_h):
    mesh = plsc.VectorSubcoreMesh(core_axis_name="cx", subcore_axis_name="sx",
                                  num_cores=1)
    kern = pl.kernel(
        _sc_cnt_body,
        out_type=jax.ShapeDtypeStruct((SEG, W), jnp.float32),
        mesh=mesh,
        compiler_params=pltpu.CompilerParams(use_tc_tiling_on_sc=False),
        scratch_types=[
            pltpu.VMEM((VC, 16), jnp.int32),
            pltpu.VMEM((VC, 16), jnp.int32),
            pltpu.VMEM((CH, 128), jnp.int32),
            pltpu.VMEM((128, 16), jnp.float32),
            pltpu.VMEM((128, 16), jnp.float32),
            pltpu.VMEM((OROWS, W), jnp.float32),
            pltpu.VMEM_SHARED((SEG, W), jnp.float32),
            pltpu.SemaphoreType.DMA,
            pltpu.SemaphoreType.DMA,
        ],
    )
    return kern(s3, p3, ones_h, zero_h)

def _sc_body(s3, p3, o3, xp, b1h, zero_h, cnt_hbm, agg_out, h_out,
             sv, pv, ov, gidx, sidx, spidx, rvidx, oidx, rows, cnrows, hbuf,
             zrow, b1v, h_sh, agg_sh,
             sem, semz, semc, sems):
    wid = lax.axis_index("sx")

    pltpu.sync_copy(s3.at[wid], sv)
    pltpu.sync_copy(p3.at[wid], pv)
    pltpu.sync_copy(o3.at[wid], ov)
    pltpu.sync_copy(b1h, b1v)
    pltpu.sync_copy(zero_h, zrow)

    zps = []
    for j in range(OROWS // 128):
        zps.append(pltpu.async_copy(
            zrow, agg_sh.at[pl.ds(wid * OROWS + j * 128, 128)], semz))
    zps.append(pltpu.async_copy(zrow, h_sh.at[pl.ds(wid * HROWS, 128)], semz))

    @plsc.parallel_loop(0, VC, unroll=8)
    def _didx(c):
        svec = sv[c]
        pvec = pv[c]
        ovec = ov[c]
        row = c >> 3
        col = (c & 7) * 16
        gidx[row, pl.ds(col, 16)] = ovec * N_RELS + pvec
        sidx[row, pl.ds(col, 16)] = svec
        spidx[row, pl.ds(col, 16)] = svec * N_RELS + pvec
        rvidx[row, pl.ds(col, 16)] = pvec * N_NODES + svec
        oidx[row, pl.ds(col, 16)] = ovec

    xp_ps = [pltpu.async_copy(xp.at[gidx.at[j]],
                              rows.at[pl.ds(j * 128, 128)], sem)
             for j in range(CH)]
    cn_ps = [pltpu.async_copy(cnt_hbm.at[rvidx.at[j]],
                              cnrows.at[pl.ds(j * 128, 128)], semc)
             for j in range(CH)]
    ob = wid * OROWS
    for cp in zps:
        cp.wait()
    plsc.subcore_barrier()

    sc_ps = []
    for j in range(CH):
        xp_ps[j].wait()
        cn_ps[j].wait()

        @plsc.parallel_loop(j * 128, (j + 1) * 128, unroll=16)
        def _dv1(i):
            rows[i] = rows[i] * cnrows[i]

        sc_ps.append(pltpu.async_copy(rows.at[pl.ds(j * 128, 128)],
                                      h_sh.at[sidx.at[j]], sems, add=True))
    for cp in sc_ps:
        cp.wait()
    plsc.subcore_barrier()

    base = wid * HROWS
    pltpu.sync_copy(h_sh.at[pl.ds(base, HROWS)], hbuf)
    b1vec = b1v[...]

    @plsc.parallel_loop(0, HROWS, unroll=4)
    def _rb(i):
        hbuf[i] = jnp.maximum(hbuf[i] + b1vec, 0.0)

    pltpu.sync_copy(hbuf, h_out.at[pl.ds(base, HROWS)])
    plsc.subcore_barrier()

    h_ps = [pltpu.async_copy(h_out.at[oidx.at[j]],
                             rows.at[pl.ds(j * 128, 128)], sem)
            for j in range(CH)]
    sc2_ps = []
    for j in range(CH):
        h_ps[j].wait()

        @plsc.parallel_loop(j * 128, (j + 1) * 128, unroll=16)
        def _dv2(i):
            rows[i] = rows[i] * cnrows[i]

        sc2_ps.append(pltpu.async_copy(rows.at[pl.ds(j * 128, 128)],
                                       agg_sh.at[spidx.at[j]], sems,
                                       add=True))
    for cp in sc2_ps:
        cp.wait()
    plsc.subcore_barrier()

    pltpu.sync_copy(agg_sh.at[pl.ds(ob, OROWS)], agg_out.at[pl.ds(ob, OROWS)])


def _sc_rgcn(s3, p3, o3, xpf, bias1, cnt_hbm):
    mesh = plsc.VectorSubcoreMesh(core_axis_name="cx", subcore_axis_name="sx",
                                  num_cores=1)
    kern = pl.kernel(
        _sc_body,
        out_type=(jax.ShapeDtypeStruct((SEG, W), jnp.float32),
                  jax.ShapeDtypeStruct((N_NODES, W), jnp.float32)),
        mesh=mesh,
        compiler_params=pltpu.CompilerParams(use_tc_tiling_on_sc=False),
        scratch_types=[
            pltpu.VMEM((VC, 16), jnp.int32),
            pltpu.VMEM((VC, 16), jnp.int32),
            pltpu.VMEM((VC, 16), jnp.int32),
            pltpu.VMEM((CH, 128), jnp.int32),
            pltpu.VMEM((CH, 128), jnp.int32),
            pltpu.VMEM((CH, 128), jnp.int32),
            pltpu.VMEM((CH, 128), jnp.int32),
            pltpu.VMEM((CH, 128), jnp.int32),
            pltpu.VMEM((TP, W), jnp.float32),
            pltpu.VMEM((TP, W), jnp.float32),
            pltpu.VMEM((HROWS, W), jnp.float32),
            pltpu.VMEM((128, 16), jnp.float32),
            pltpu.VMEM((16,), jnp.float32),
            pltpu.VMEM_SHARED((N_NODES, W), jnp.float32),
            pltpu.VMEM_SHARED((SEG, W), jnp.float32),
            pltpu.SemaphoreType.DMA,
            pltpu.SemaphoreType.DMA,
            pltpu.SemaphoreType.DMA,
            pltpu.SemaphoreType.DMA,
        ],
    )
    zero_h = jnp.zeros((128, 16), jnp.float32)
    return kern(s3, p3, o3, xpf, bias1, zero_h, cnt_hbm)


def kernel(triples, node_embeddings, weights1, weights2, bias1, bias2):
    s = triples[:, 0].astype(jnp.int32)
    p = triples[:, 1].astype(jnp.int32)
    o = triples[:, 2].astype(jnp.int32)
    s3 = s.reshape(NT, VC, 16)
    p3 = p.reshape(NT, VC, 16)
    o3 = o.reshape(NT, VC, 16)

    ones_h = jnp.ones((128, 16), jnp.float32)
    zero_h = jnp.zeros((128, 16), jnp.float32)
    cnt = _sc_cnt(s3, p3, ones_h, zero_h)

    w1cat = jnp.transpose(weights1, (1, 0, 2)).reshape(EMB, RW)
    xproj = _mm1(node_embeddings, w1cat)
    xpf = xproj.reshape(SEG, W)

    agg, _ = _sc_rgcn(s3, p3, o3, xpf, bias1, cnt)

    a2 = agg.reshape(N_NODES, RW)
    w2r = weights2.reshape(RW, N_CLASSES)
    return _mm2(a2, w2r, bias2.reshape(1, N_CLASSES))


# speedup vs baseline: 1.0109x; 1.0109x over previous
import jax
import jax.numpy as jnp
from jax import lax
from jax.experimental import pallas as pl
from jax.experimental.pallas import tpu as pltpu
from jax.experimental.pallas import tpu_sc as plsc

N_NODES = 2048
N_RELS = 8
N_CLASSES = 2048
EMB = 1600
W = 16
RW = N_RELS * W
T = 32768
SEG = N_RELS * N_NODES

NT = 16
TP = T // NT
VC = TP // 16
CH = TP // 128
HROWS = N_NODES // NT
OROWS = SEG // NT


def _mm1_body(x_ref, w_ref, o_ref):
    o_ref[...] = jnp.dot(x_ref[...], w_ref[...],
                         preferred_element_type=jnp.float32)


def _mm1(x, w1cat):
    return pl.pallas_call(
        _mm1_body,
        grid=(8,),
        in_specs=[
            pl.BlockSpec((N_NODES // 8, EMB), lambda i: (i, 0)),
            pl.BlockSpec((EMB, RW), lambda i: (0, 0)),
        ],
        out_specs=pl.BlockSpec((N_NODES // 8, RW), lambda i: (i, 0)),
        out_shape=jax.ShapeDtypeStruct((N_NODES, RW), jnp.float32),
    )(x, w1cat)


def _mm2_body(a_ref, b_ref, bias_ref, o_ref):
    o_ref[...] = (jnp.dot(a_ref[...], b_ref[...],
                          preferred_element_type=jnp.float32)
                  + bias_ref[...])


def _mm2(a, b, bias2_row):
    return pl.pallas_call(
        _mm2_body,
        grid=(8,),
        in_specs=[
            pl.BlockSpec((N_NODES // 8, RW), lambda i: (i, 0)),
            pl.BlockSpec((RW, N_CLASSES), lambda i: (0, 0)),
            pl.BlockSpec((1, N_CLASSES), lambda i: (0, 0)),
        ],
        out_specs=pl.BlockSpec((N_NODES // 8, N_CLASSES), lambda i: (i, 0)),
        out_shape=jax.ShapeDtypeStruct((N_NODES, N_CLASSES), jnp.float32),
    )(a, b, bias2_row)


def _sc_cnt_body(s3, p3, ones_h, zero_h, cnt_out,
                 sv, pv, rvidx, ones, zrow, cnt_sh, semz, semc):
    wid = lax.axis_index("sx")
    pltpu.sync_copy(s3.at[wid], sv)
    pltpu.sync_copy(p3.at[wid], pv)
    pltpu.sync_copy(ones_h, ones)
    pltpu.sync_copy(zero_h, zrow)

    zps = []
    for j in range(OROWS // 128):
        zps.append(pltpu.async_copy(
            zrow, cnt_sh.at[pl.ds(wid * OROWS + j * 128, 128)], semz))

    @plsc.parallel_loop(0, VC, unroll=4)
    def _didx(c):
        rvidx[c >> 3, pl.ds((c & 7) * 16, 16)] = pv[c] * N_NODES + sv[c]

    for cp in zps:
        cp.wait()
    plsc.subcore_barrier()

    cnt_ps = [pltpu.async_copy(ones, cnt_sh.at[rvidx.at[j]], semc, add=True)
              for j in range(CH)]
    for cp in cnt_ps:
        cp.wait()
    plsc.subcore_barrier()

    ob = wid * OROWS
    pltpu.sync_copy(cnt_sh.at[pl.ds(ob, OROWS)], cnt_out.at[pl.ds(ob, OROWS)])


def _sc_cnt(s3, p3, ones_h, zero_h):
    mesh = plsc.VectorSubcoreMesh(core_axis_name="cx", subcore_axis_name="sx",
                                  num_cores=1)
    kern = pl.kernel(
        _sc_cnt_body,
        out_type=jax.ShapeDtypeStruct((SEG, W), jnp.float32),
        mesh=mesh,
        compiler_params=pltpu.CompilerParams(use_tc_tiling_on_sc=False),
        scratch_types=[
            pltpu.VMEM((VC, 16), jnp.int32),
            pltpu.VMEM((VC, 16), jnp.int32),
            pltpu.VMEM((CH, 128), jnp.int32),
            pltpu.VMEM((128, 16), jnp.float32),
            pltpu.VMEM((128, 16), jnp.float32),
            pltpu.VMEM_SHARED((SEG, W), jnp.float32),
            pltpu.SemaphoreType.DMA,
            pltpu.SemaphoreType.DMA,
        ],
    )
    return kern(s3, p3, ones_h, zero_h)

def _sc_body(s3, p3, o3, xp, b1h, zero_h, cnt_hbm, agg_out, h_out,
             sv, pv, ov, gidx, sidx, spidx, rvidx, oidx, rows, cnrows, hbuf,
             zrow, b1v, h_sh, agg_sh,
             sem, semz, semc, sems):
    wid = lax.axis_index("sx")

    pltpu.sync_copy(s3.at[wid], sv)
    pltpu.sync_copy(p3.at[wid], pv)
    pltpu.sync_copy(o3.at[wid], ov)
    pltpu.sync_copy(b1h, b1v)
    pltpu.sync_copy(zero_h, zrow)

    zps = []
    for j in range(OROWS // 128):
        zps.append(pltpu.async_copy(
            zrow, agg_sh.at[pl.ds(wid * OROWS + j * 128, 128)], semz))
    zps.append(pltpu.async_copy(zrow, h_sh.at[pl.ds(wid * HROWS, 128)], semz))

    @plsc.parallel_loop(0, VC, unroll=4)
    def _didx(c):
        svec = sv[c]
        pvec = pv[c]
        ovec = ov[c]
        row = c >> 3
        col = (c & 7) * 16
        gidx[row, pl.ds(col, 16)] = ovec * N_RELS + pvec
        sidx[row, pl.ds(col, 16)] = svec
        spidx[row, pl.ds(col, 16)] = svec * N_RELS + pvec
        rvidx[row, pl.ds(col, 16)] = pvec * N_NODES + svec
        oidx[row, pl.ds(col, 16)] = ovec

    xp_ps = [pltpu.async_copy(xp.at[gidx.at[j]],
                              rows.at[pl.ds(j * 128, 128)], sem)
             for j in range(CH)]
    cn_ps = [pltpu.async_copy(cnt_hbm.at[rvidx.at[j]],
                              cnrows.at[pl.ds(j * 128, 128)], semc)
             for j in range(CH)]
    ob = wid * OROWS
    for cp in zps:
        cp.wait()
    plsc.subcore_barrier()

    sc_ps = []
    for j in range(CH):
        xp_ps[j].wait()
        cn_ps[j].wait()

        @plsc.parallel_loop(j * 128, (j + 1) * 128, unroll=8)
        def _dv1(i):
            rows[i] = rows[i] / cnrows[i]

        sc_ps.append(pltpu.async_copy(rows.at[pl.ds(j * 128, 128)],
                                      h_sh.at[sidx.at[j]], sems, add=True))
    for cp in sc_ps:
        cp.wait()
    plsc.subcore_barrier()

    base = wid * HROWS
    pltpu.sync_copy(h_sh.at[pl.ds(base, HROWS)], hbuf)
    b1vec = b1v[...]

    @plsc.parallel_loop(0, HROWS, unroll=4)
    def _rb(i):
        hbuf[i] = jnp.maximum(hbuf[i] + b1vec, 0.0)

    pltpu.sync_copy(hbuf, h_out.at[pl.ds(base, HROWS)])
    plsc.subcore_barrier()

    h_ps = [pltpu.async_copy(h_out.at[oidx.at[j]],
                             rows.at[pl.ds(j * 128, 128)], sem)
            for j in range(CH)]
    sc2_ps = []
    for j in range(CH):
        h_ps[j].wait()

        @plsc.parallel_loop(j * 128, (j + 1) * 128, unroll=8)
        def _dv2(i):
            rows[i] = rows[i] / cnrows[i]

        sc2_ps.append(pltpu.async_copy(rows.at[pl.ds(j * 128, 128)],
                                       agg_sh.at[spidx.at[j]], sems,
                                       add=True))
    for cp in sc2_ps:
        cp.wait()
    plsc.subcore_barrier()

    pltpu.sync_copy(agg_sh.at[pl.ds(ob, OROWS)], agg_out.at[pl.ds(ob, OROWS)])


def _sc_rgcn(s3, p3, o3, xpf, bias1, cnt_hbm):
    mesh = plsc.VectorSubcoreMesh(core_axis_name="cx", subcore_axis_name="sx",
                                  num_cores=1)
    kern = pl.kernel(
        _sc_body,
        out_type=(jax.ShapeDtypeStruct((SEG, W), jnp.float32),
                  jax.ShapeDtypeStruct((N_NODES, W), jnp.float32)),
        mesh=mesh,
        compiler_params=pltpu.CompilerParams(use_tc_tiling_on_sc=False),
        scratch_types=[
            pltpu.VMEM((VC, 16), jnp.int32),
            pltpu.VMEM((VC, 16), jnp.int32),
            pltpu.VMEM((VC, 16), jnp.int32),
            pltpu.VMEM((CH, 128), jnp.int32),
            pltpu.VMEM((CH, 128), jnp.int32),
            pltpu.VMEM((CH, 128), jnp.int32),
            pltpu.VMEM((CH, 128), jnp.int32),
            pltpu.VMEM((CH, 128), jnp.int32),
            pltpu.VMEM((TP, W), jnp.float32),
            pltpu.VMEM((TP, W), jnp.float32),
            pltpu.VMEM((HROWS, W), jnp.float32),
            pltpu.VMEM((128, 16), jnp.float32),
            pltpu.VMEM((16,), jnp.float32),
            pltpu.VMEM_SHARED((N_NODES, W), jnp.float32),
            pltpu.VMEM_SHARED((SEG, W), jnp.float32),
            pltpu.SemaphoreType.DMA,
            pltpu.SemaphoreType.DMA,
            pltpu.SemaphoreType.DMA,
            pltpu.SemaphoreType.DMA,
        ],
    )
    zero_h = jnp.zeros((128, 16), jnp.float32)
    return kern(s3, p3, o3, xpf, bias1, zero_h, cnt_hbm)


def kernel(triples, node_embeddings, weights1, weights2, bias1, bias2):
    s = triples[:, 0].astype(jnp.int32)
    p = triples[:, 1].astype(jnp.int32)
    o = triples[:, 2].astype(jnp.int32)
    s3 = s.reshape(NT, VC, 16)
    p3 = p.reshape(NT, VC, 16)
    o3 = o.reshape(NT, VC, 16)

    ones_h = jnp.ones((128, 16), jnp.float32)
    zero_h = jnp.zeros((128, 16), jnp.float32)
    cnt = _sc_cnt(s3, p3, ones_h, zero_h)

    w1cat = jnp.transpose(weights1, (1, 0, 2)).reshape(EMB, RW)
    xproj = _mm1(node_embeddings, w1cat)
    xpf = xproj.reshape(SEG, W)

    agg, _ = _sc_rgcn(s3, p3, o3, xpf, bias1, cnt)

    a2 = agg.reshape(N_NODES, RW)
    w2r = weights2.reshape(RW, N_CLASSES)
    return _mm2(a2, w2r, bias2.reshape(1, N_CLASSES))
